# baseline (speedup 1.0000x reference)
import functools

import jax
import jax.numpy as jnp
import numpy as np
from jax.experimental import pallas as pl
from jax.experimental.pallas import tpu as pltpu

N = 10000
E = 160000
NIN = 5
MSG = 128
UPD = 128
NGRAPH = 64
HS = 64
BNC = 1.0 / np.sqrt(1.0 + 1e-3)
TRANS = jnp.array([0.0, 0.0, -200.0, 10000.0, 0.0], dtype=jnp.float32)
SCALE = jnp.array([100.0, 100.0, 100.0, 2500.0, 0.25], dtype=jnp.float32)

EBLK = 2000


def _edge_mlp_body(gi_ref, gj_ref, w2_ref, b2_ref, out_ref):
    m = jnp.maximum(gi_ref[...] + gj_ref[...], 0.0)
    m2 = jnp.dot(m, w2_ref[...], preferred_element_type=jnp.float32)
    out_ref[...] = jnp.maximum(m2 + b2_ref[...], 0.0)


def _edge_mlp(gi, gj, w2, b2):
    nb = E // EBLK
    return pl.pallas_call(
        _edge_mlp_body,
        grid=(nb,),
        in_specs=[
            pl.BlockSpec((EBLK, MSG), lambda i: (i, 0)),
            pl.BlockSpec((EBLK, MSG), lambda i: (i, 0)),
            pl.BlockSpec((MSG, MSG), lambda i: (0, 0)),
            pl.BlockSpec((1, MSG), lambda i: (0, 0)),
        ],
        out_specs=pl.BlockSpec((EBLK, MSG), lambda i: (i, 0)),
        out_shape=jax.ShapeDtypeStruct((E, MSG), jnp.float32),
    )(gi, gj, w2, b2.reshape(1, MSG))


def kernel(x, edge_index, graph_ids, params):
    relu = jax.nn.relu
    xx = (x - TRANS) / SCALE
    idx_i = edge_index[:, 0]
    idx_j = edge_index[:, 1]
    cnt = jax.ops.segment_sum(jnp.ones((E, 1), jnp.float32), idx_i, num_segments=N)
    h = xx
    for lp in params["mp"]:
        din = lp["Wm1"].shape[0]
        hd = (din - NIN) // 2
        Wa = lp["Wm1"][:hd]
        Wb = lp["Wm1"][hd:2 * hd]
        Wc = lp["Wm1"][2 * hd:]
        xc = BNC * (xx @ Wc)
        P = h @ Wa - xc + lp["bm1"]
        Q = h @ Wb + xc
        gi = P[idx_i]
        gj = Q[idx_j]
        m = _edge_mlp(gi, gj, lp["Wm2"], lp["bm2"])
        mn = jax.ops.segment_min(m, idx_i, num_segments=N)
        mx = jax.ops.segment_max(m, idx_i, num_segments=N)
        mean = jax.ops.segment_sum(m, idx_i, num_segments=N) / cnt
        mean2 = jax.ops.segment_sum(m * m, idx_i, num_segments=N) / cnt
        var = mean2 - mean ** 2
        emb = jnp.concatenate([mn, mx, mean, var], axis=1)
        u = relu(emb @ lp["Wu1"] + lp["bu1"])
        u = relu(u @ lp["Wu2"] + lp["bu2"])
        h = u * BNC
    p1 = jax.ops.segment_max(h, graph_ids, num_segments=NGRAPH)
    p3 = jax.ops.segment_sum(h, graph_ids, num_segments=NGRAPH)
    gcnt = jax.ops.segment_sum(jnp.ones((N, 1), jnp.float32), graph_ids, num_segments=NGRAPH)
    p2 = p3 / gcnt
    p4 = -jax.ops.segment_max(-h, graph_ids, num_segments=NGRAPH)
    g = jnp.concatenate([p1, p2, p3, p4], axis=1)
    for (W, b) in params["dec"]:
        g = (g @ W + b) * BNC
    outs = []
    for sp in params["split"]:
        (W0, b0), (W1, b1), (W2, b2) = sp
        t = (g @ W0 + b0) * BNC
        t = (t @ W1 + b1) * BNC
        t = t @ W2 + b2
        outs.append(t)
    o = jnp.concatenate(outs, axis=1)
    nrm = jnp.linalg.norm(o[:, :3], axis=1, keepdims=True)
    denom = jnp.where(nrm > 0, nrm, 1.0)
    dirv = jnp.where(nrm > 0, o[:, :3] / denom, 0.0)
    return jnp.concatenate([dirv, o[:, 3:4]], axis=1)


# speedup vs baseline: 1.1779x; 1.1779x over previous
import functools

import jax
import jax.numpy as jnp
import numpy as np
from jax import lax
from jax.experimental import pallas as pl
from jax.experimental.pallas import tpu as pltpu
from jax.experimental.pallas import tpu_sc as plsc

N = 10000
E = 160000
NIN = 5
MSG = 128
UPD = 128
NGRAPH = 64
HS = 64
BNC = 1.0 / np.sqrt(1.0 + 1e-3)
TRANS = jnp.array([0.0, 0.0, -200.0, 10000.0, 0.0], dtype=jnp.float32)
SCALE = jnp.array([100.0, 100.0, 100.0, 2500.0, 0.25], dtype=jnp.float32)

EBLK = 2000

_NC = 2
_NS = 16
_NW = _NC * _NS
_EPW = E // _NW
_GCH = 128
_NFULL = _EPW // _GCH
_SC_MESH = plsc.VectorSubcoreMesh(core_axis_name="c", subcore_axis_name="s")


@functools.partial(
    pl.kernel,
    out_type=[jax.ShapeDtypeStruct((E, MSG), jnp.float32),
              jax.ShapeDtypeStruct((E, MSG), jnp.float32)],
    mesh=_SC_MESH,
    scratch_types=[
        pltpu.VMEM((_GCH,), jnp.int32),
        pltpu.VMEM((_GCH,), jnp.int32),
        pltpu.VMEM((_GCH, MSG), jnp.float32),
        pltpu.VMEM((_GCH, MSG), jnp.float32),
        pltpu.SemaphoreType.DMA,
        pltpu.SemaphoreType.DMA,
    ],
)
def _sc_gather(p_hbm, q_hbm, ii_hbm, jj_hbm, gi_hbm, gj_hbm,
               idxi, idxj, rowsi, rowsj, semi, semj):
    wid = lax.axis_index("s") * _NC + lax.axis_index("c")
    base0 = wid * _EPW

    def body(c, carry):
        base = base0 + lax.min(c * _GCH, _EPW - _GCH)
        pltpu.sync_copy(ii_hbm.at[pl.ds(base, _GCH)], idxi)
        pltpu.sync_copy(jj_hbm.at[pl.ds(base, _GCH)], idxj)
        cpi = pltpu.async_copy(p_hbm.at[idxi], rowsi, semi)
        cpj = pltpu.async_copy(q_hbm.at[idxj], rowsj, semj)
        cpi.wait()
        cpj.wait()
        pltpu.sync_copy(rowsi, gi_hbm.at[pl.ds(base, _GCH)])
        pltpu.sync_copy(rowsj, gj_hbm.at[pl.ds(base, _GCH)])
        return carry

    lax.fori_loop(0, _NFULL + 1, body, 0)


def _edge_mlp_body(gi_ref, gj_ref, w2_ref, b2_ref, out_ref):
    m = jnp.maximum(gi_ref[...] + gj_ref[...], 0.0)
    m2 = jnp.dot(m, w2_ref[...], preferred_element_type=jnp.float32)
    out_ref[...] = jnp.maximum(m2 + b2_ref[...], 0.0)


def _edge_mlp(gi, gj, w2, b2):
    nb = E // EBLK
    return pl.pallas_call(
        _edge_mlp_body,
        grid=(nb,),
        in_specs=[
            pl.BlockSpec((EBLK, MSG), lambda i: (i, 0)),
            pl.BlockSpec((EBLK, MSG), lambda i: (i, 0)),
            pl.BlockSpec((MSG, MSG), lambda i: (0, 0)),
            pl.BlockSpec((1, MSG), lambda i: (0, 0)),
        ],
        out_specs=pl.BlockSpec((EBLK, MSG), lambda i: (i, 0)),
        out_shape=jax.ShapeDtypeStruct((E, MSG), jnp.float32),
    )(gi, gj, w2, b2.reshape(1, MSG))


def kernel(x, edge_index, graph_ids, params):
    relu = jax.nn.relu
    xx = (x - TRANS) / SCALE
    idx_i = edge_index[:, 0]
    idx_j = edge_index[:, 1]
    cnt = jax.ops.segment_sum(jnp.ones((E, 1), jnp.float32), idx_i, num_segments=N)
    h = xx
    for lp in params["mp"]:
        din = lp["Wm1"].shape[0]
        hd = (din - NIN) // 2
        Wa = lp["Wm1"][:hd]
        Wb = lp["Wm1"][hd:2 * hd]
        Wc = lp["Wm1"][2 * hd:]
        xc = BNC * (xx @ Wc)
        P = h @ Wa - xc + lp["bm1"]
        Q = h @ Wb + xc
        gi, gj = _sc_gather(P, Q, idx_i, idx_j)
        m = _edge_mlp(gi, gj, lp["Wm2"], lp["bm2"])
        mn = jax.ops.segment_min(m, idx_i, num_segments=N)
        mx = jax.ops.segment_max(m, idx_i, num_segments=N)
        mean = jax.ops.segment_sum(m, idx_i, num_segments=N) / cnt
        mean2 = jax.ops.segment_sum(m * m, idx_i, num_segments=N) / cnt
        var = mean2 - mean ** 2
        emb = jnp.concatenate([mn, mx, mean, var], axis=1)
        u = relu(emb @ lp["Wu1"] + lp["bu1"])
        u = relu(u @ lp["Wu2"] + lp["bu2"])
        h = u * BNC
    p1 = jax.ops.segment_max(h, graph_ids, num_segments=NGRAPH)
    p3 = jax.ops.segment_sum(h, graph_ids, num_segments=NGRAPH)
    gcnt = jax.ops.segment_sum(jnp.ones((N, 1), jnp.float32), graph_ids, num_segments=NGRAPH)
    p2 = p3 / gcnt
    p4 = -jax.ops.segment_max(-h, graph_ids, num_segments=NGRAPH)
    g = jnp.concatenate([p1, p2, p3, p4], axis=1)
    for (W, b) in params["dec"]:
        g = (g @ W + b) * BNC
    outs = []
    for sp in params["split"]:
        (W0, b0), (W1, b1), (W2, b2) = sp
        t = (g @ W0 + b0) * BNC
        t = (t @ W1 + b1) * BNC
        t = t @ W2 + b2
        outs.append(t)
    o = jnp.concatenate(outs, axis=1)
    nrm = jnp.linalg.norm(o[:, :3], axis=1, keepdims=True)
    denom = jnp.where(nrm > 0, nrm, 1.0)
    dirv = jnp.where(nrm > 0, o[:, :3] / denom, 0.0)
    return jnp.concatenate([dirv, o[:, 3:4]], axis=1)
